# Initial kernel scaffold; baseline (speedup 1.0000x reference)
import jax
import jax.numpy as jnp
from jax.experimental import pallas as pl


def kernel(keys, queue, ptr):
    raise NotImplementedError("write your pallas kernel here")


# speedup vs baseline: 8.0493x; 8.0493x over previous
import functools

import jax
import jax.numpy as jnp
from jax.experimental import pallas as pl
from jax.experimental.pallas import tpu as pltpu

_BLK = 2048


def _enqueue_block(sc_ref, keys_pad_ref, queue_ref, out_ref, *, n, kq, blk):
    i = pl.program_id(0)
    r0 = i * blk
    p = sc_ref[0]
    base = sc_ref[1]
    q_mod = jax.lax.rem(r0 - p + kq, kq)
    q = jnp.where(q_mod < n, q_mod, q_mod - kq)
    offset = (jnp.maximum(base + q, 0) // 8) * 8
    aligned = keys_pad_ref[pl.ds(offset, blk), :]
    u = jax.lax.broadcasted_iota(jnp.int32, (blk, 1), 0)
    ki = q + u
    mask = (ki >= 0) & (ki < n)
    out_ref[...] = jnp.where(mask, aligned, queue_ref[...])


def kernel(keys, queue, ptr):
    n, d = keys.shape
    kq = queue.shape[0]
    blk = _BLK
    p = jnp.asarray(ptr, jnp.int32) % kq
    base = blk + p % 8
    pad_rows = n + 2 * blk + 8
    keys_pad = jax.lax.dynamic_update_slice(
        jnp.zeros((pad_rows, d), keys.dtype), keys, (base, jnp.int32(0))
    )
    grid_spec = pltpu.PrefetchScalarGridSpec(
        num_scalar_prefetch=1,
        grid=(kq // blk,),
        in_specs=[
            pl.BlockSpec((pad_rows, d), lambda i, pref: (0, 0)),
            pl.BlockSpec((blk, d), lambda i, pref: (i, 0)),
        ],
        out_specs=pl.BlockSpec((blk, d), lambda i, pref: (i, 0)),
    )
    return pl.pallas_call(
        functools.partial(_enqueue_block, n=n, kq=kq, blk=blk),
        grid_spec=grid_spec,
        out_shape=jax.ShapeDtypeStruct((kq, d), queue.dtype),
    )(jnp.stack([p, base]), keys_pad, queue)


# speedup vs baseline: 8.3169x; 1.0332x over previous
import functools

import jax
import jax.numpy as jnp
from jax.experimental import pallas as pl
from jax.experimental.pallas import tpu as pltpu

_BLK = 4096


def _enqueue_block(sc_ref, keys_pad_ref, queue_ref, out_ref, *, n, kq, blk):
    i = pl.program_id(0)
    r0 = i * blk
    p = sc_ref[0]
    base = sc_ref[1]
    q_mod = jax.lax.rem(r0 - p + kq, kq)
    q = jnp.where(q_mod < n, q_mod, q_mod - kq)
    offset = (jnp.maximum(base + q, 0) // 8) * 8
    aligned = keys_pad_ref[pl.ds(offset, blk), :]
    u = jax.lax.broadcasted_iota(jnp.int32, (blk, 1), 0)
    ki = q + u
    mask = (ki >= 0) & (ki < n)
    out_ref[...] = jnp.where(mask, aligned, queue_ref[...])


def kernel(keys, queue, ptr):
    n, d = keys.shape
    kq = queue.shape[0]
    blk = _BLK
    p = jnp.asarray(ptr, jnp.int32) % kq
    base = blk + p % 8
    pad_rows = n + 2 * blk + 8
    keys_pad = jax.lax.dynamic_update_slice(
        jnp.zeros((pad_rows, d), keys.dtype), keys, (base, jnp.int32(0))
    )
    grid_spec = pltpu.PrefetchScalarGridSpec(
        num_scalar_prefetch=1,
        grid=(kq // blk,),
        in_specs=[
            pl.BlockSpec((pad_rows, d), lambda i, pref: (0, 0)),
            pl.BlockSpec((blk, d), lambda i, pref: (i, 0)),
        ],
        out_specs=pl.BlockSpec((blk, d), lambda i, pref: (i, 0)),
    )
    return pl.pallas_call(
        functools.partial(_enqueue_block, n=n, kq=kq, blk=blk),
        grid_spec=grid_spec,
        out_shape=jax.ShapeDtypeStruct((kq, d), queue.dtype),
    )(jnp.stack([p, base]), keys_pad, queue)
